# Initial kernel scaffold; baseline (speedup 1.0000x reference)
import jax
import jax.numpy as jnp
from jax.experimental import pallas as pl


def kernel(x, edge_index, batch, W1, b1, W2, b2, Wfc, bfc):
    raise NotImplementedError("write your pallas kernel here")


# speedup vs baseline: 9.8015x; 9.8015x over previous
import functools

import jax
import jax.numpy as jnp
from jax import lax
from jax.experimental import pallas as pl
from jax.experimental.pallas import tpu as pltpu
from jax.experimental.pallas import tpu_sc as plsc

N = 10000
E = 320000
D = 128
G = 128
CLS = 10

NC = 2
NS = 16
NW = NC * NS
EPW = E // NW
CH = 80
NCHUNK = EPW // CH
NPAD = 10240
RPT = NPAD // NS

RB = 80
NRB = N // RB


def _deg_body(dst_hbm, ones_hbm, zeros_hbm, out_hbm, didx, ones_rows, acc):
    cid = lax.axis_index("c")
    sid = lax.axis_index("s")
    wid = sid * NC + cid

    pltpu.sync_copy(ones_hbm, ones_rows)
    pltpu.sync_copy(zeros_hbm, acc.at[pl.ds(sid * RPT, RPT)])
    plsc.subcore_barrier()

    base0 = wid * EPW

    def cbody(k, c):
        pltpu.sync_copy(dst_hbm.at[pl.ds(base0 + k * CH, CH)], didx)
        pltpu.sync_copy(ones_rows, acc.at[didx], add=True)
        return c

    lax.fori_loop(0, NCHUNK, cbody, 0)
    plsc.subcore_barrier()
    pltpu.sync_copy(acc.at[pl.ds(sid * RPT, RPT)],
                    out_hbm.at[pl.ds(cid * NPAD + sid * RPT, RPT)])


def _deg_kernel(dst, ones, zeros):
    fn = pl.kernel(
        _deg_body,
        mesh=plsc.VectorSubcoreMesh(core_axis_name="c", subcore_axis_name="s"),
        out_type=jax.ShapeDtypeStruct((NC * NPAD, D), jnp.float32),
        scratch_types=[
            pltpu.VMEM((CH,), jnp.int32),
            pltpu.VMEM((CH, D), jnp.float32),
            pltpu.VMEM_SHARED((NPAD, D), jnp.float32),
        ],
    )
    return fn(dst, ones, zeros)


def _agg_body(hs_hbm, src_hbm, dst_hbm, zeros_hbm, out_hbm, sidx, didx, rows,
              acc, sem):
    cid = lax.axis_index("c")
    sid = lax.axis_index("s")
    wid = sid * NC + cid

    pltpu.sync_copy(zeros_hbm, acc.at[pl.ds(sid * RPT, RPT)])
    plsc.subcore_barrier()

    base0 = wid * EPW

    def cbody(k, c):
        b = base0 + k * CH
        pltpu.sync_copy(src_hbm.at[pl.ds(b, CH)], sidx)
        pltpu.sync_copy(dst_hbm.at[pl.ds(b, CH)], didx)
        pltpu.async_copy(hs_hbm.at[sidx], rows, sem).wait()
        pltpu.sync_copy(rows, acc.at[didx], add=True)
        return c

    lax.fori_loop(0, NCHUNK, cbody, 0)
    plsc.subcore_barrier()
    pltpu.sync_copy(acc.at[pl.ds(sid * RPT, RPT)],
                    out_hbm.at[pl.ds(cid * NPAD + sid * RPT, RPT)])


def _agg_kernel(hs, src, dst, zeros):
    fn = pl.kernel(
        _agg_body,
        mesh=plsc.VectorSubcoreMesh(core_axis_name="c", subcore_axis_name="s"),
        out_type=jax.ShapeDtypeStruct((NC * NPAD, D), jnp.float32),
        scratch_types=[
            pltpu.VMEM((CH,), jnp.int32),
            pltpu.VMEM((CH,), jnp.int32),
            pltpu.VMEM((CH, D), jnp.float32),
            pltpu.VMEM_SHARED((NPAD, D), jnp.float32),
            pltpu.SemaphoreType.DMA,
        ],
    )
    return fn(hs, src, dst, zeros)


def _mm1_body(x_ref, w_ref, dpa_ref, dpb_ref, hs_ref, dv_ref):
    dv = lax.rsqrt(dpa_ref[...] + dpb_ref[...] + 1.0)
    dv_ref[...] = dv
    h = jnp.dot(x_ref[...], w_ref[...], preferred_element_type=jnp.float32)
    hs_ref[...] = h * dv


def _mm1(x, W1, dpa, dpb):
    return pl.pallas_call(
        _mm1_body,
        grid=(NRB,),
        in_specs=[
            pl.BlockSpec((RB, D), lambda i: (i, 0)),
            pl.BlockSpec((D, D), lambda i: (0, 0)),
            pl.BlockSpec((RB, D), lambda i: (i, 0)),
            pl.BlockSpec((RB, D), lambda i: (i, 0)),
        ],
        out_specs=[
            pl.BlockSpec((RB, D), lambda i: (i, 0)),
            pl.BlockSpec((RB, D), lambda i: (i, 0)),
        ],
        out_shape=[
            jax.ShapeDtypeStruct((N, D), jnp.float32),
            jax.ShapeDtypeStruct((N, D), jnp.float32),
        ],
    )(x, W1, dpa, dpb)


def _mid_body(a_ref, b_ref, hs_ref, dv_ref, b1_ref, w2_ref, out_ref):
    t = dv_ref[...] * (a_ref[...] + b_ref[...] + hs_ref[...]) + b1_ref[...]
    h = jnp.maximum(t, 0.0)
    out_ref[...] = jnp.dot(h, w2_ref[...],
                           preferred_element_type=jnp.float32) * dv_ref[...]


def _mid(s1a, s1b, hs1, dinv2, b1, W2):
    return pl.pallas_call(
        _mid_body,
        grid=(NRB,),
        in_specs=[
            pl.BlockSpec((RB, D), lambda i: (i, 0)),
            pl.BlockSpec((RB, D), lambda i: (i, 0)),
            pl.BlockSpec((RB, D), lambda i: (i, 0)),
            pl.BlockSpec((RB, D), lambda i: (i, 0)),
            pl.BlockSpec((1, D), lambda i: (0, 0)),
            pl.BlockSpec((D, D), lambda i: (0, 0)),
        ],
        out_specs=pl.BlockSpec((RB, D), lambda i: (i, 0)),
        out_shape=jax.ShapeDtypeStruct((N, D), jnp.float32),
    )(s1a, s1b, hs1, dinv2, b1, W2)


def _final_body(a_ref, b_ref, hs_ref, dv_ref, b2_ref, bf_ref, wfc_ref, bfc_ref,
                out_ref, sumsT, cnts):
    i = pl.program_id(0)

    @pl.when(i == 0)
    def _():
        sumsT[...] = jnp.zeros((D, G), jnp.float32)
        cnts[...] = jnp.zeros((1, G), jnp.float32)

    t = dv_ref[...] * (a_ref[...] + b_ref[...] + hs_ref[...]) + b2_ref[...]
    h = jnp.maximum(t, 0.0)
    bval = bf_ref[...]
    gid = lax.broadcasted_iota(jnp.int32, (RB, G), 1).astype(jnp.float32)
    oh = jnp.where(bval == gid, 1.0, 0.0)
    sumsT[...] += lax.dot_general(h, oh, (((0,), (0,)), ((), ())),
                                  preferred_element_type=jnp.float32)
    cnts[...] += lax.dot_general(jnp.ones((1, RB), jnp.float32), oh,
                                 (((1,), (0,)), ((), ())),
                                 preferred_element_type=jnp.float32)

    @pl.when(i == NRB - 1)
    def _():
        embT = sumsT[...] / jnp.maximum(cnts[...], 1.0)
        out_ref[...] = lax.dot_general(embT, wfc_ref[...],
                                       (((0,), (0,)), ((), ())),
                                       preferred_element_type=jnp.float32
                                       ) + bfc_ref[...]


def _final(s2a, s2b, hs2, dinv2, b2, batchf, Wfc, bfc):
    return pl.pallas_call(
        _final_body,
        grid=(NRB,),
        in_specs=[
            pl.BlockSpec((RB, D), lambda i: (i, 0)),
            pl.BlockSpec((RB, D), lambda i: (i, 0)),
            pl.BlockSpec((RB, D), lambda i: (i, 0)),
            pl.BlockSpec((RB, D), lambda i: (i, 0)),
            pl.BlockSpec((1, D), lambda i: (0, 0)),
            pl.BlockSpec((RB, G), lambda i: (i, 0)),
            pl.BlockSpec((D, CLS), lambda i: (0, 0)),
            pl.BlockSpec((1, CLS), lambda i: (0, 0)),
        ],
        out_specs=pl.BlockSpec((G, CLS), lambda i: (0, 0)),
        out_shape=jax.ShapeDtypeStruct((G, CLS), jnp.float32),
        scratch_shapes=[
            pltpu.VMEM((D, G), jnp.float32),
            pltpu.VMEM((1, G), jnp.float32),
        ],
    )(s2a, s2b, hs2, dinv2, b2, batchf, Wfc, bfc)


def kernel(x, edge_index, batch, W1, b1, W2, b2, Wfc, bfc):
    ei = edge_index.astype(jnp.int32)
    src = ei[0]
    dst = ei[1]
    batchf = jnp.broadcast_to(batch.astype(jnp.float32)[:, None], (N, G))
    zeros = jnp.zeros((RPT, D), jnp.float32)
    ones = jnp.ones((CH, D), jnp.float32)

    dp = _deg_kernel(dst, ones, zeros)
    hs1, dinv2 = _mm1(x, W1, dp[:N], dp[NPAD:NPAD + N])
    s1 = _agg_kernel(hs1, src, dst, zeros)
    hs2 = _mid(s1[:N], s1[NPAD:NPAD + N], hs1, dinv2, b1.reshape(1, D), W2)
    s2 = _agg_kernel(hs2, src, dst, zeros)
    return _final(s2[:N], s2[NPAD:NPAD + N], hs2, dinv2, b2.reshape(1, D),
                  batchf, Wfc, bfc.reshape(1, CLS))


# speedup vs baseline: 14.2131x; 1.4501x over previous
import functools

import jax
import jax.numpy as jnp
from jax import lax
from jax.experimental import pallas as pl
from jax.experimental.pallas import tpu as pltpu
from jax.experimental.pallas import tpu_sc as plsc

N = 10000
E = 320000
D = 128
G = 128
CLS = 10

NC = 2
NS = 16
NW = NC * NS
EPW = E // NW
CH = 80
NCHUNK = EPW // CH
GCH = 128
EPT = 10240
EPAD = EPT * NW
NCH2 = EPT // GCH
NPAD = 10240
RPT = NPAD // NS

RB = 80
NRB = N // RB


def _deg_body(dst_hbm, ones_hbm, zeros_hbm, out_hbm, didx, ones_rows, acc):
    cid = lax.axis_index("c")
    sid = lax.axis_index("s")
    wid = sid * NC + cid

    pltpu.sync_copy(ones_hbm, ones_rows)
    pltpu.sync_copy(zeros_hbm, acc.at[pl.ds(sid * RPT, RPT)])
    plsc.subcore_barrier()

    base0 = wid * EPW

    def cbody(k, c):
        pltpu.sync_copy(dst_hbm.at[pl.ds(base0 + k * CH, CH)], didx)
        pltpu.sync_copy(ones_rows, acc.at[didx], add=True)
        return c

    lax.fori_loop(0, NCHUNK, cbody, 0)
    plsc.subcore_barrier()
    pltpu.sync_copy(acc.at[pl.ds(sid * RPT, RPT)],
                    out_hbm.at[pl.ds(cid * NPAD + sid * RPT, RPT)])


def _deg_kernel(dst, ones, zeros):
    fn = pl.kernel(
        _deg_body,
        mesh=plsc.VectorSubcoreMesh(core_axis_name="c", subcore_axis_name="s"),
        out_type=jax.ShapeDtypeStruct((NC * NPAD, D), jnp.float32),
        scratch_types=[
            pltpu.VMEM((CH,), jnp.int32),
            pltpu.VMEM((CH, D), jnp.float32),
            pltpu.VMEM_SHARED((NPAD, D), jnp.float32),
        ],
    )
    return fn(dst, ones, zeros)


def _agg_body(hs_hbm, srcp_hbm, dstp_hbm, zeros_hbm, out_hbm, sa0, da0, sa1,
              da1, rows0, rows1, acc, sg0, sg1):
    cid = lax.axis_index("c")
    sid = lax.axis_index("s")
    wid = sid * NC + cid

    pltpu.sync_copy(zeros_hbm, acc.at[pl.ds(sid * RPT, RPT)])
    plsc.subcore_barrier()

    base0 = wid * NCH2

    pltpu.sync_copy(srcp_hbm.at[base0], sa0)
    pltpu.sync_copy(dstp_hbm.at[base0], da0)
    pltpu.async_copy(hs_hbm.at[sa0], rows0, sg0)
    pltpu.sync_copy(srcp_hbm.at[base0 + 1], sa1)
    pltpu.sync_copy(dstp_hbm.at[base0 + 1], da1)
    pltpu.async_copy(hs_hbm.at[sa1], rows1, sg1)

    def body(g, c):
        k0 = 2 * g
        pltpu.make_async_copy(hs_hbm.at[sa0], rows0, sg0).wait()
        pltpu.sync_copy(rows0, acc.at[da0], add=True)
        pltpu.sync_copy(srcp_hbm.at[base0 + k0 + 2], sa0)
        pltpu.sync_copy(dstp_hbm.at[base0 + k0 + 2], da0)
        pltpu.async_copy(hs_hbm.at[sa0], rows0, sg0)
        pltpu.make_async_copy(hs_hbm.at[sa1], rows1, sg1).wait()
        pltpu.sync_copy(rows1, acc.at[da1], add=True)
        pltpu.sync_copy(srcp_hbm.at[base0 + k0 + 3], sa1)
        pltpu.sync_copy(dstp_hbm.at[base0 + k0 + 3], da1)
        pltpu.async_copy(hs_hbm.at[sa1], rows1, sg1)
        return c

    lax.fori_loop(0, NCH2 // 2 - 1, body, 0)

    pltpu.make_async_copy(hs_hbm.at[sa0], rows0, sg0).wait()
    pltpu.sync_copy(rows0, acc.at[da0], add=True)
    pltpu.make_async_copy(hs_hbm.at[sa1], rows1, sg1).wait()
    pltpu.sync_copy(rows1, acc.at[da1], add=True)

    plsc.subcore_barrier()
    pltpu.sync_copy(acc.at[pl.ds(sid * RPT, RPT)],
                    out_hbm.at[pl.ds(cid * NPAD + sid * RPT, RPT)])


def _agg_kernel(hs, srcp, dstp, zeros):
    fn = pl.kernel(
        _agg_body,
        mesh=plsc.VectorSubcoreMesh(core_axis_name="c", subcore_axis_name="s"),
        out_type=jax.ShapeDtypeStruct((NC * NPAD, D), jnp.float32),
        scratch_types=[
            pltpu.VMEM((GCH,), jnp.int32),
            pltpu.VMEM((GCH,), jnp.int32),
            pltpu.VMEM((GCH,), jnp.int32),
            pltpu.VMEM((GCH,), jnp.int32),
            pltpu.VMEM((GCH, D), jnp.float32),
            pltpu.VMEM((GCH, D), jnp.float32),
            pltpu.VMEM_SHARED((NPAD, D), jnp.float32),
            pltpu.SemaphoreType.DMA,
            pltpu.SemaphoreType.DMA,
        ],
    )
    return fn(hs, srcp, dstp, zeros)


def _mm1_body(x_ref, w_ref, dpa_ref, dpb_ref, hs_ref, dv_ref):
    dv = lax.rsqrt(dpa_ref[...] + dpb_ref[...] + 1.0)
    dv_ref[...] = dv
    h = jnp.dot(x_ref[...], w_ref[...], preferred_element_type=jnp.float32)
    hs_ref[...] = h * dv


def _mm1(x, W1, dpa, dpb):
    return pl.pallas_call(
        _mm1_body,
        grid=(NRB,),
        in_specs=[
            pl.BlockSpec((RB, D), lambda i: (i, 0)),
            pl.BlockSpec((D, D), lambda i: (0, 0)),
            pl.BlockSpec((RB, D), lambda i: (i, 0)),
            pl.BlockSpec((RB, D), lambda i: (i, 0)),
        ],
        out_specs=[
            pl.BlockSpec((RB, D), lambda i: (i, 0)),
            pl.BlockSpec((RB, D), lambda i: (i, 0)),
        ],
        out_shape=[
            jax.ShapeDtypeStruct((N, D), jnp.float32),
            jax.ShapeDtypeStruct((N, D), jnp.float32),
        ],
    )(x, W1, dpa, dpb)


def _mid_body(a_ref, b_ref, hs_ref, dv_ref, b1_ref, w2_ref, out_ref):
    t = dv_ref[...] * (a_ref[...] + b_ref[...] + hs_ref[...]) + b1_ref[...]
    h = jnp.maximum(t, 0.0)
    out_ref[...] = jnp.dot(h, w2_ref[...],
                           preferred_element_type=jnp.float32) * dv_ref[...]


def _mid(s1a, s1b, hs1, dinv2, b1, W2):
    return pl.pallas_call(
        _mid_body,
        grid=(NRB,),
        in_specs=[
            pl.BlockSpec((RB, D), lambda i: (i, 0)),
            pl.BlockSpec((RB, D), lambda i: (i, 0)),
            pl.BlockSpec((RB, D), lambda i: (i, 0)),
            pl.BlockSpec((RB, D), lambda i: (i, 0)),
            pl.BlockSpec((1, D), lambda i: (0, 0)),
            pl.BlockSpec((D, D), lambda i: (0, 0)),
        ],
        out_specs=pl.BlockSpec((RB, D), lambda i: (i, 0)),
        out_shape=jax.ShapeDtypeStruct((N, D), jnp.float32),
    )(s1a, s1b, hs1, dinv2, b1, W2)


def _final_body(a_ref, b_ref, hs_ref, dv_ref, b2_ref, bf_ref, wfc_ref, bfc_ref,
                out_ref, sumsT, cnts):
    i = pl.program_id(0)

    @pl.when(i == 0)
    def _():
        sumsT[...] = jnp.zeros((D, G), jnp.float32)
        cnts[...] = jnp.zeros((1, G), jnp.float32)

    t = dv_ref[...] * (a_ref[...] + b_ref[...] + hs_ref[...]) + b2_ref[...]
    h = jnp.maximum(t, 0.0)
    bval = bf_ref[...]
    gid = lax.broadcasted_iota(jnp.int32, (RB, G), 1).astype(jnp.float32)
    oh = jnp.where(bval == gid, 1.0, 0.0)
    sumsT[...] += lax.dot_general(h, oh, (((0,), (0,)), ((), ())),
                                  preferred_element_type=jnp.float32)
    cnts[...] += lax.dot_general(jnp.ones((1, RB), jnp.float32), oh,
                                 (((1,), (0,)), ((), ())),
                                 preferred_element_type=jnp.float32)

    @pl.when(i == NRB - 1)
    def _():
        embT = sumsT[...] / jnp.maximum(cnts[...], 1.0)
        out_ref[...] = lax.dot_general(embT, wfc_ref[...],
                                       (((0,), (0,)), ((), ())),
                                       preferred_element_type=jnp.float32
                                       ) + bfc_ref[...]


def _final(s2a, s2b, hs2, dinv2, b2, batchf, Wfc, bfc):
    return pl.pallas_call(
        _final_body,
        grid=(NRB,),
        in_specs=[
            pl.BlockSpec((RB, D), lambda i: (i, 0)),
            pl.BlockSpec((RB, D), lambda i: (i, 0)),
            pl.BlockSpec((RB, D), lambda i: (i, 0)),
            pl.BlockSpec((RB, D), lambda i: (i, 0)),
            pl.BlockSpec((1, D), lambda i: (0, 0)),
            pl.BlockSpec((RB, G), lambda i: (i, 0)),
            pl.BlockSpec((D, CLS), lambda i: (0, 0)),
            pl.BlockSpec((1, CLS), lambda i: (0, 0)),
        ],
        out_specs=pl.BlockSpec((G, CLS), lambda i: (0, 0)),
        out_shape=jax.ShapeDtypeStruct((G, CLS), jnp.float32),
        scratch_shapes=[
            pltpu.VMEM((D, G), jnp.float32),
            pltpu.VMEM((1, G), jnp.float32),
        ],
    )(s2a, s2b, hs2, dinv2, b2, batchf, Wfc, bfc)


def kernel(x, edge_index, batch, W1, b1, W2, b2, Wfc, bfc):
    ei = edge_index.astype(jnp.int32)
    src = ei[0]
    dst = ei[1]
    batchf = jnp.broadcast_to(batch.astype(jnp.float32)[:, None], (N, G))
    zeros = jnp.zeros((RPT, D), jnp.float32)
    ones = jnp.ones((CH, D), jnp.float32)

    dp = _deg_kernel(dst, ones, zeros)
    hs1, dinv2 = _mm1(x, W1, dp[:N], dp[NPAD:NPAD + N])
    pad = EPAD - E
    fill = jnp.arange(pad, dtype=jnp.int32)
    srcp = jnp.concatenate([src, fill % N]).reshape(NW * NCH2, GCH)
    dstp = jnp.concatenate([dst, N + fill % (NPAD - N)]).reshape(NW * NCH2, GCH)
    s1 = _agg_kernel(hs1, srcp, dstp, zeros)
    hs2 = _mid(s1[:N], s1[NPAD:NPAD + N], hs1, dinv2, b1.reshape(1, D), W2)
    s2 = _agg_kernel(hs2, srcp, dstp, zeros)
    return _final(s2[:N], s2[NPAD:NPAD + N], hs2, dinv2, b2.reshape(1, D),
                  batchf, Wfc, bfc.reshape(1, CLS))


# speedup vs baseline: 14.5067x; 1.0207x over previous
import functools

import jax
import jax.numpy as jnp
from jax import lax
from jax.experimental import pallas as pl
from jax.experimental.pallas import tpu as pltpu
from jax.experimental.pallas import tpu_sc as plsc

N = 10000
E = 320000
D = 128
G = 128
CLS = 10

NC = 2
NS = 16
NW = NC * NS
EPW = E // NW
CH = 80
NCHUNK = EPW // CH
GCH = 128
EPT = 10240
EPAD = EPT * NW
NCH2 = EPT // GCH
NPAD = 10240
RPT = NPAD // NS

RB = 80
NRB = N // RB


def _deg_body(dstp_hbm, ones_hbm, zeros_hbm, out_hbm, da0, da1, ones_rows,
              acc, ss0, ss1):
    cid = lax.axis_index("c")
    sid = lax.axis_index("s")
    wid = sid * NC + cid

    pltpu.sync_copy(ones_hbm, ones_rows)
    pltpu.sync_copy(zeros_hbm, acc.at[pl.ds(sid * RPT, RPT)])
    plsc.subcore_barrier()

    base0 = wid * NCH2
    pltpu.sync_copy(dstp_hbm.at[base0], da0)
    pltpu.sync_copy(dstp_hbm.at[base0 + 1], da1)

    def cbody(g, c):
        k0 = 2 * g
        pltpu.async_copy(ones_rows, acc.at[da0], ss0, add=True)
        pltpu.async_copy(ones_rows, acc.at[da1], ss1, add=True)
        pltpu.make_async_copy(ones_rows, acc.at[da0], ss0).wait()
        pltpu.sync_copy(dstp_hbm.at[base0 + k0 + 2], da0)
        pltpu.make_async_copy(ones_rows, acc.at[da1], ss1).wait()
        pltpu.sync_copy(dstp_hbm.at[base0 + k0 + 3], da1)
        return c

    lax.fori_loop(0, NCH2 // 2 - 1, cbody, 0)
    pltpu.sync_copy(ones_rows, acc.at[da0], add=True)
    pltpu.sync_copy(ones_rows, acc.at[da1], add=True)
    plsc.subcore_barrier()
    pltpu.sync_copy(acc.at[pl.ds(sid * RPT, RPT)],
                    out_hbm.at[pl.ds(cid * NPAD + sid * RPT, RPT)])


def _deg_kernel(dstp, ones, zeros):
    fn = pl.kernel(
        _deg_body,
        mesh=plsc.VectorSubcoreMesh(core_axis_name="c", subcore_axis_name="s"),
        out_type=jax.ShapeDtypeStruct((NC * NPAD, D), jnp.float32),
        scratch_types=[
            pltpu.VMEM((GCH,), jnp.int32),
            pltpu.VMEM((GCH,), jnp.int32),
            pltpu.VMEM((GCH, D), jnp.float32),
            pltpu.VMEM_SHARED((NPAD, D), jnp.float32),
            pltpu.SemaphoreType.DMA,
            pltpu.SemaphoreType.DMA,
        ],
    )
    return fn(dstp, ones, zeros)


def _agg_body(hs_hbm, srcp_hbm, dstp_hbm, zeros_hbm, out_hbm, sa0, da0, sa1,
              da1, rows0, rows1, acc, sg0, sg1, ss0, ss1):
    cid = lax.axis_index("c")
    sid = lax.axis_index("s")
    wid = sid * NC + cid

    pltpu.sync_copy(zeros_hbm, acc.at[pl.ds(sid * RPT, RPT)])
    plsc.subcore_barrier()

    base0 = wid * NCH2

    pltpu.sync_copy(srcp_hbm.at[base0], sa0)
    pltpu.sync_copy(dstp_hbm.at[base0], da0)
    pltpu.async_copy(hs_hbm.at[sa0], rows0, sg0)
    pltpu.sync_copy(srcp_hbm.at[base0 + 1], sa1)
    pltpu.sync_copy(dstp_hbm.at[base0 + 1], da1)
    pltpu.async_copy(hs_hbm.at[sa1], rows1, sg1)

    def body(g, c):
        k0 = 2 * g
        pltpu.make_async_copy(hs_hbm.at[sa0], rows0, sg0).wait()
        pltpu.async_copy(rows0, acc.at[da0], ss0, add=True)
        pltpu.make_async_copy(hs_hbm.at[sa1], rows1, sg1).wait()
        pltpu.async_copy(rows1, acc.at[da1], ss1, add=True)
        pltpu.make_async_copy(rows0, acc.at[da0], ss0).wait()
        pltpu.sync_copy(srcp_hbm.at[base0 + k0 + 2], sa0)
        pltpu.sync_copy(dstp_hbm.at[base0 + k0 + 2], da0)
        pltpu.async_copy(hs_hbm.at[sa0], rows0, sg0)
        pltpu.make_async_copy(rows1, acc.at[da1], ss1).wait()
        pltpu.sync_copy(srcp_hbm.at[base0 + k0 + 3], sa1)
        pltpu.sync_copy(dstp_hbm.at[base0 + k0 + 3], da1)
        pltpu.async_copy(hs_hbm.at[sa1], rows1, sg1)
        return c

    lax.fori_loop(0, NCH2 // 2 - 1, body, 0)

    pltpu.make_async_copy(hs_hbm.at[sa0], rows0, sg0).wait()
    pltpu.sync_copy(rows0, acc.at[da0], add=True)
    pltpu.make_async_copy(hs_hbm.at[sa1], rows1, sg1).wait()
    pltpu.sync_copy(rows1, acc.at[da1], add=True)

    plsc.subcore_barrier()
    pltpu.sync_copy(acc.at[pl.ds(sid * RPT, RPT)],
                    out_hbm.at[pl.ds(cid * NPAD + sid * RPT, RPT)])


def _agg_kernel(hs, srcp, dstp, zeros):
    fn = pl.kernel(
        _agg_body,
        mesh=plsc.VectorSubcoreMesh(core_axis_name="c", subcore_axis_name="s"),
        out_type=jax.ShapeDtypeStruct((NC * NPAD, D), jnp.float32),
        scratch_types=[
            pltpu.VMEM((GCH,), jnp.int32),
            pltpu.VMEM((GCH,), jnp.int32),
            pltpu.VMEM((GCH,), jnp.int32),
            pltpu.VMEM((GCH,), jnp.int32),
            pltpu.VMEM((GCH, D), jnp.float32),
            pltpu.VMEM((GCH, D), jnp.float32),
            pltpu.VMEM_SHARED((NPAD, D), jnp.float32),
            pltpu.SemaphoreType.DMA,
            pltpu.SemaphoreType.DMA,
            pltpu.SemaphoreType.DMA,
            pltpu.SemaphoreType.DMA,
        ],
    )
    return fn(hs, srcp, dstp, zeros)


def _mm1_body(x_ref, w_ref, dpa_ref, dpb_ref, hs_ref, dv_ref):
    dv = lax.rsqrt(dpa_ref[...] + dpb_ref[...] + 1.0)
    dv_ref[...] = dv
    h = jnp.dot(x_ref[...], w_ref[...], preferred_element_type=jnp.float32)
    hs_ref[...] = h * dv


def _mm1(x, W1, dpa, dpb):
    return pl.pallas_call(
        _mm1_body,
        grid=(NRB,),
        in_specs=[
            pl.BlockSpec((RB, D), lambda i: (i, 0)),
            pl.BlockSpec((D, D), lambda i: (0, 0)),
            pl.BlockSpec((RB, D), lambda i: (i, 0)),
            pl.BlockSpec((RB, D), lambda i: (i, 0)),
        ],
        out_specs=[
            pl.BlockSpec((RB, D), lambda i: (i, 0)),
            pl.BlockSpec((RB, D), lambda i: (i, 0)),
        ],
        out_shape=[
            jax.ShapeDtypeStruct((N, D), jnp.float32),
            jax.ShapeDtypeStruct((N, D), jnp.float32),
        ],
    )(x, W1, dpa, dpb)


def _mid_body(a_ref, b_ref, hs_ref, dv_ref, b1_ref, w2_ref, out_ref):
    t = dv_ref[...] * (a_ref[...] + b_ref[...] + hs_ref[...]) + b1_ref[...]
    h = jnp.maximum(t, 0.0)
    out_ref[...] = jnp.dot(h, w2_ref[...],
                           preferred_element_type=jnp.float32) * dv_ref[...]


def _mid(s1a, s1b, hs1, dinv2, b1, W2):
    return pl.pallas_call(
        _mid_body,
        grid=(NRB,),
        in_specs=[
            pl.BlockSpec((RB, D), lambda i: (i, 0)),
            pl.BlockSpec((RB, D), lambda i: (i, 0)),
            pl.BlockSpec((RB, D), lambda i: (i, 0)),
            pl.BlockSpec((RB, D), lambda i: (i, 0)),
            pl.BlockSpec((1, D), lambda i: (0, 0)),
            pl.BlockSpec((D, D), lambda i: (0, 0)),
        ],
        out_specs=pl.BlockSpec((RB, D), lambda i: (i, 0)),
        out_shape=jax.ShapeDtypeStruct((N, D), jnp.float32),
    )(s1a, s1b, hs1, dinv2, b1, W2)


def _final_body(a_ref, b_ref, hs_ref, dv_ref, b2_ref, bf_ref, wfc_ref, bfc_ref,
                out_ref, sumsT, cnts):
    i = pl.program_id(0)

    @pl.when(i == 0)
    def _():
        sumsT[...] = jnp.zeros((D, G), jnp.float32)
        cnts[...] = jnp.zeros((1, G), jnp.float32)

    t = dv_ref[...] * (a_ref[...] + b_ref[...] + hs_ref[...]) + b2_ref[...]
    h = jnp.maximum(t, 0.0)
    bval = bf_ref[...]
    gid = lax.broadcasted_iota(jnp.int32, (RB, G), 1).astype(jnp.float32)
    oh = jnp.where(bval == gid, 1.0, 0.0)
    sumsT[...] += lax.dot_general(h, oh, (((0,), (0,)), ((), ())),
                                  preferred_element_type=jnp.float32)
    cnts[...] += lax.dot_general(jnp.ones((1, RB), jnp.float32), oh,
                                 (((1,), (0,)), ((), ())),
                                 preferred_element_type=jnp.float32)

    @pl.when(i == NRB - 1)
    def _():
        embT = sumsT[...] / jnp.maximum(cnts[...], 1.0)
        out_ref[...] = lax.dot_general(embT, wfc_ref[...],
                                       (((0,), (0,)), ((), ())),
                                       preferred_element_type=jnp.float32
                                       ) + bfc_ref[...]


def _final(s2a, s2b, hs2, dinv2, b2, batchf, Wfc, bfc):
    return pl.pallas_call(
        _final_body,
        grid=(NRB,),
        in_specs=[
            pl.BlockSpec((RB, D), lambda i: (i, 0)),
            pl.BlockSpec((RB, D), lambda i: (i, 0)),
            pl.BlockSpec((RB, D), lambda i: (i, 0)),
            pl.BlockSpec((RB, D), lambda i: (i, 0)),
            pl.BlockSpec((1, D), lambda i: (0, 0)),
            pl.BlockSpec((RB, G), lambda i: (i, 0)),
            pl.BlockSpec((D, CLS), lambda i: (0, 0)),
            pl.BlockSpec((1, CLS), lambda i: (0, 0)),
        ],
        out_specs=pl.BlockSpec((G, CLS), lambda i: (0, 0)),
        out_shape=jax.ShapeDtypeStruct((G, CLS), jnp.float32),
        scratch_shapes=[
            pltpu.VMEM((D, G), jnp.float32),
            pltpu.VMEM((1, G), jnp.float32),
        ],
    )(s2a, s2b, hs2, dinv2, b2, batchf, Wfc, bfc)


def kernel(x, edge_index, batch, W1, b1, W2, b2, Wfc, bfc):
    ei = edge_index.astype(jnp.int32)
    src = ei[0]
    dst = ei[1]
    batchf = jnp.broadcast_to(batch.astype(jnp.float32)[:, None], (N, G))
    zeros = jnp.zeros((RPT, D), jnp.float32)
    ones = jnp.ones((GCH, D), jnp.float32)
    pad = EPAD - E
    fill = jnp.arange(pad, dtype=jnp.int32)
    srcp = jnp.concatenate([src, fill % N]).reshape(NW * NCH2, GCH)
    dstp = jnp.concatenate([dst, N + fill % (NPAD - N)]).reshape(NW * NCH2, GCH)

    dp = _deg_kernel(dstp, ones, zeros)
    hs1, dinv2 = _mm1(x, W1, dp[:N], dp[NPAD:NPAD + N])
    s1 = _agg_kernel(hs1, srcp, dstp, zeros)
    hs2 = _mid(s1[:N], s1[NPAD:NPAD + N], hs1, dinv2, b1.reshape(1, D), W2)
    s2 = _agg_kernel(hs2, srcp, dstp, zeros)
    return _final(s2[:N], s2[NPAD:NPAD + N], hs2, dinv2, b2.reshape(1, D),
                  batchf, Wfc, bfc.reshape(1, CLS))


# speedup vs baseline: 19.6524x; 1.3547x over previous
import functools

import jax
import jax.numpy as jnp
from jax import lax
from jax.experimental import pallas as pl
from jax.experimental.pallas import tpu as pltpu
from jax.experimental.pallas import tpu_sc as plsc

N = 10000
E = 320000
D = 128
G = 128
CLS = 10

NC = 2
NS = 16
NW = NC * NS
EPW = E // NW
CH = 80
NCHUNK = EPW // CH
GCH = 128
EPT = 10240
EPAD = EPT * NW
NCH2 = EPT // GCH
NPAD = 10240
RPT = NPAD // NS

RB = 1000
NRB = N // RB


def _deg_body(dstp_hbm, ones_hbm, zeros_hbm, out_hbm, da0, da1, ones_rows,
              acc, ss0, ss1):
    cid = lax.axis_index("c")
    sid = lax.axis_index("s")
    wid = sid * NC + cid

    pltpu.sync_copy(ones_hbm, ones_rows)
    pltpu.sync_copy(zeros_hbm, acc.at[pl.ds(sid * RPT, RPT)])
    plsc.subcore_barrier()

    base0 = wid * NCH2
    pltpu.sync_copy(dstp_hbm.at[base0], da0)
    pltpu.sync_copy(dstp_hbm.at[base0 + 1], da1)

    def cbody(g, c):
        k0 = 2 * g
        pltpu.async_copy(ones_rows, acc.at[da0], ss0, add=True)
        pltpu.async_copy(ones_rows, acc.at[da1], ss1, add=True)
        pltpu.make_async_copy(ones_rows, acc.at[da0], ss0).wait()
        pltpu.sync_copy(dstp_hbm.at[base0 + k0 + 2], da0)
        pltpu.make_async_copy(ones_rows, acc.at[da1], ss1).wait()
        pltpu.sync_copy(dstp_hbm.at[base0 + k0 + 3], da1)
        return c

    lax.fori_loop(0, NCH2 // 2 - 1, cbody, 0)
    pltpu.sync_copy(ones_rows, acc.at[da0], add=True)
    pltpu.sync_copy(ones_rows, acc.at[da1], add=True)
    plsc.subcore_barrier()
    pltpu.sync_copy(acc.at[pl.ds(sid * RPT, RPT)],
                    out_hbm.at[pl.ds(cid * NPAD + sid * RPT, RPT)])


def _deg_kernel(dstp, ones, zeros):
    fn = pl.kernel(
        _deg_body,
        mesh=plsc.VectorSubcoreMesh(core_axis_name="c", subcore_axis_name="s"),
        out_type=jax.ShapeDtypeStruct((NC * NPAD, D), jnp.float32),
        scratch_types=[
            pltpu.VMEM((GCH,), jnp.int32),
            pltpu.VMEM((GCH,), jnp.int32),
            pltpu.VMEM((GCH, D), jnp.float32),
            pltpu.VMEM_SHARED((NPAD, D), jnp.float32),
            pltpu.SemaphoreType.DMA,
            pltpu.SemaphoreType.DMA,
        ],
    )
    return fn(dstp, ones, zeros)


def _agg_body(hs_hbm, srcp_hbm, dstp_hbm, zeros_hbm, out_hbm, sa0, da0, sa1,
              da1, rows0, rows1, acc, sg0, sg1, ss0, ss1):
    cid = lax.axis_index("c")
    sid = lax.axis_index("s")
    wid = sid * NC + cid

    pltpu.sync_copy(zeros_hbm, acc.at[pl.ds(sid * RPT, RPT)])
    plsc.subcore_barrier()

    base0 = wid * NCH2

    pltpu.sync_copy(srcp_hbm.at[base0], sa0)
    pltpu.sync_copy(dstp_hbm.at[base0], da0)
    pltpu.async_copy(hs_hbm.at[sa0], rows0, sg0)
    pltpu.sync_copy(srcp_hbm.at[base0 + 1], sa1)
    pltpu.sync_copy(dstp_hbm.at[base0 + 1], da1)
    pltpu.async_copy(hs_hbm.at[sa1], rows1, sg1)

    def body(g, c):
        k0 = 2 * g
        pltpu.make_async_copy(hs_hbm.at[sa0], rows0, sg0).wait()
        pltpu.async_copy(rows0, acc.at[da0], ss0, add=True)
        pltpu.make_async_copy(hs_hbm.at[sa1], rows1, sg1).wait()
        pltpu.async_copy(rows1, acc.at[da1], ss1, add=True)
        pltpu.make_async_copy(rows0, acc.at[da0], ss0).wait()
        pltpu.sync_copy(srcp_hbm.at[base0 + k0 + 2], sa0)
        pltpu.sync_copy(dstp_hbm.at[base0 + k0 + 2], da0)
        pltpu.async_copy(hs_hbm.at[sa0], rows0, sg0)
        pltpu.make_async_copy(rows1, acc.at[da1], ss1).wait()
        pltpu.sync_copy(srcp_hbm.at[base0 + k0 + 3], sa1)
        pltpu.sync_copy(dstp_hbm.at[base0 + k0 + 3], da1)
        pltpu.async_copy(hs_hbm.at[sa1], rows1, sg1)
        return c

    lax.fori_loop(0, NCH2 // 2 - 1, body, 0)

    pltpu.make_async_copy(hs_hbm.at[sa0], rows0, sg0).wait()
    pltpu.sync_copy(rows0, acc.at[da0], add=True)
    pltpu.make_async_copy(hs_hbm.at[sa1], rows1, sg1).wait()
    pltpu.sync_copy(rows1, acc.at[da1], add=True)

    plsc.subcore_barrier()
    pltpu.sync_copy(acc.at[pl.ds(sid * RPT, RPT)],
                    out_hbm.at[pl.ds(cid * NPAD + sid * RPT, RPT)])


def _agg_kernel(hs, srcp, dstp, zeros):
    fn = pl.kernel(
        _agg_body,
        mesh=plsc.VectorSubcoreMesh(core_axis_name="c", subcore_axis_name="s"),
        out_type=jax.ShapeDtypeStruct((NC * NPAD, D), jnp.float32),
        scratch_types=[
            pltpu.VMEM((GCH,), jnp.int32),
            pltpu.VMEM((GCH,), jnp.int32),
            pltpu.VMEM((GCH,), jnp.int32),
            pltpu.VMEM((GCH,), jnp.int32),
            pltpu.VMEM((GCH, D), jnp.float32),
            pltpu.VMEM((GCH, D), jnp.float32),
            pltpu.VMEM_SHARED((NPAD, D), jnp.float32),
            pltpu.SemaphoreType.DMA,
            pltpu.SemaphoreType.DMA,
            pltpu.SemaphoreType.DMA,
            pltpu.SemaphoreType.DMA,
        ],
    )
    return fn(hs, srcp, dstp, zeros)


def _mm1_body(x_ref, w_ref, dpa_ref, dpb_ref, hs_ref, dv_ref):
    dv = lax.rsqrt(dpa_ref[...] + dpb_ref[...] + 1.0)
    dv_ref[...] = dv
    h = jnp.dot(x_ref[...], w_ref[...], preferred_element_type=jnp.float32)
    hs_ref[...] = h * dv


def _mm1(x, W1, dpa, dpb):
    return pl.pallas_call(
        _mm1_body,
        grid=(NRB,),
        in_specs=[
            pl.BlockSpec((RB, D), lambda i: (i, 0)),
            pl.BlockSpec((D, D), lambda i: (0, 0)),
            pl.BlockSpec((RB, D), lambda i: (i, 0)),
            pl.BlockSpec((RB, D), lambda i: (i, 0)),
        ],
        out_specs=[
            pl.BlockSpec((RB, D), lambda i: (i, 0)),
            pl.BlockSpec((RB, D), lambda i: (i, 0)),
        ],
        out_shape=[
            jax.ShapeDtypeStruct((N, D), jnp.float32),
            jax.ShapeDtypeStruct((N, D), jnp.float32),
        ],
    )(x, W1, dpa, dpb)


def _mid_body(a_ref, b_ref, hs_ref, dv_ref, b1_ref, w2_ref, out_ref):
    t = dv_ref[...] * (a_ref[...] + b_ref[...] + hs_ref[...]) + b1_ref[...]
    h = jnp.maximum(t, 0.0)
    out_ref[...] = jnp.dot(h, w2_ref[...],
                           preferred_element_type=jnp.float32) * dv_ref[...]


def _mid(s1a, s1b, hs1, dinv2, b1, W2):
    return pl.pallas_call(
        _mid_body,
        grid=(NRB,),
        in_specs=[
            pl.BlockSpec((RB, D), lambda i: (i, 0)),
            pl.BlockSpec((RB, D), lambda i: (i, 0)),
            pl.BlockSpec((RB, D), lambda i: (i, 0)),
            pl.BlockSpec((RB, D), lambda i: (i, 0)),
            pl.BlockSpec((1, D), lambda i: (0, 0)),
            pl.BlockSpec((D, D), lambda i: (0, 0)),
        ],
        out_specs=pl.BlockSpec((RB, D), lambda i: (i, 0)),
        out_shape=jax.ShapeDtypeStruct((N, D), jnp.float32),
    )(s1a, s1b, hs1, dinv2, b1, W2)


def _final_body(a_ref, b_ref, hs_ref, dv_ref, b2_ref, bf_ref, wfc_ref, bfc_ref,
                out_ref, sumsT, cnts):
    i = pl.program_id(0)

    @pl.when(i == 0)
    def _():
        sumsT[...] = jnp.zeros((D, G), jnp.float32)
        cnts[...] = jnp.zeros((1, G), jnp.float32)

    t = dv_ref[...] * (a_ref[...] + b_ref[...] + hs_ref[...]) + b2_ref[...]
    h = jnp.maximum(t, 0.0)
    bval = bf_ref[...]
    gid = lax.broadcasted_iota(jnp.int32, (RB, G), 1).astype(jnp.float32)
    oh = jnp.where(bval == gid, 1.0, 0.0)
    sumsT[...] += lax.dot_general(h, oh, (((0,), (0,)), ((), ())),
                                  preferred_element_type=jnp.float32)
    cnts[...] += lax.dot_general(jnp.ones((1, RB), jnp.float32), oh,
                                 (((1,), (0,)), ((), ())),
                                 preferred_element_type=jnp.float32)

    @pl.when(i == NRB - 1)
    def _():
        embT = sumsT[...] / jnp.maximum(cnts[...], 1.0)
        out_ref[...] = lax.dot_general(embT, wfc_ref[...],
                                       (((0,), (0,)), ((), ())),
                                       preferred_element_type=jnp.float32
                                       ) + bfc_ref[...]


def _final(s2a, s2b, hs2, dinv2, b2, batchf, Wfc, bfc):
    return pl.pallas_call(
        _final_body,
        grid=(NRB,),
        in_specs=[
            pl.BlockSpec((RB, D), lambda i: (i, 0)),
            pl.BlockSpec((RB, D), lambda i: (i, 0)),
            pl.BlockSpec((RB, D), lambda i: (i, 0)),
            pl.BlockSpec((RB, D), lambda i: (i, 0)),
            pl.BlockSpec((1, D), lambda i: (0, 0)),
            pl.BlockSpec((RB, G), lambda i: (i, 0)),
            pl.BlockSpec((D, CLS), lambda i: (0, 0)),
            pl.BlockSpec((1, CLS), lambda i: (0, 0)),
        ],
        out_specs=pl.BlockSpec((G, CLS), lambda i: (0, 0)),
        out_shape=jax.ShapeDtypeStruct((G, CLS), jnp.float32),
        scratch_shapes=[
            pltpu.VMEM((D, G), jnp.float32),
            pltpu.VMEM((1, G), jnp.float32),
        ],
    )(s2a, s2b, hs2, dinv2, b2, batchf, Wfc, bfc)


def kernel(x, edge_index, batch, W1, b1, W2, b2, Wfc, bfc):
    ei = edge_index.astype(jnp.int32)
    src = ei[0]
    dst = ei[1]
    batchf = jnp.broadcast_to(batch.astype(jnp.float32)[:, None], (N, G))
    zeros = jnp.zeros((RPT, D), jnp.float32)
    ones = jnp.ones((GCH, D), jnp.float32)
    pad = EPAD - E
    fill = jnp.arange(pad, dtype=jnp.int32)
    srcp = jnp.concatenate([src, fill % N]).reshape(NW * NCH2, GCH)
    dstp = jnp.concatenate([dst, N + fill % (NPAD - N)]).reshape(NW * NCH2, GCH)

    dp = _deg_kernel(dstp, ones, zeros)
    hs1, dinv2 = _mm1(x, W1, dp[:N], dp[NPAD:NPAD + N])
    s1 = _agg_kernel(hs1, srcp, dstp, zeros)
    hs2 = _mid(s1[:N], s1[NPAD:NPAD + N], hs1, dinv2, b1.reshape(1, D), W2)
    s2 = _agg_kernel(hs2, srcp, dstp, zeros)
    return _final(s2[:N], s2[NPAD:NPAD + N], hs2, dinv2, b2.reshape(1, D),
                  batchf, Wfc, bfc.reshape(1, CLS))


# speedup vs baseline: 19.8304x; 1.0091x over previous
import functools

import jax
import jax.numpy as jnp
from jax import lax
from jax.experimental import pallas as pl
from jax.experimental.pallas import tpu as pltpu
from jax.experimental.pallas import tpu_sc as plsc

N = 10000
E = 320000
D = 128
G = 128
CLS = 10

NC = 2
NS = 16
NW = NC * NS
EPW = E // NW
CH = 80
NCHUNK = EPW // CH
GCH = 128
EPT = 10240
EPAD = EPT * NW
NCH2 = EPT // GCH
NPAD = 10240
RPT = NPAD // NS

RB = 2000
NRB = N // RB


def _deg_body(dstp_hbm, ones_hbm, zeros_hbm, out_hbm, da0, da1, ones_rows,
              acc, ss0, ss1):
    cid = lax.axis_index("c")
    sid = lax.axis_index("s")
    wid = sid * NC + cid

    pltpu.sync_copy(ones_hbm, ones_rows)
    pltpu.sync_copy(zeros_hbm, acc.at[pl.ds(sid * RPT, RPT)])
    plsc.subcore_barrier()

    base0 = wid * NCH2
    pltpu.sync_copy(dstp_hbm.at[base0], da0)
    pltpu.sync_copy(dstp_hbm.at[base0 + 1], da1)

    def cbody(g, c):
        k0 = 2 * g
        pltpu.async_copy(ones_rows, acc.at[da0], ss0, add=True)
        pltpu.async_copy(ones_rows, acc.at[da1], ss1, add=True)
        pltpu.make_async_copy(ones_rows, acc.at[da0], ss0).wait()
        pltpu.sync_copy(dstp_hbm.at[base0 + k0 + 2], da0)
        pltpu.make_async_copy(ones_rows, acc.at[da1], ss1).wait()
        pltpu.sync_copy(dstp_hbm.at[base0 + k0 + 3], da1)
        return c

    lax.fori_loop(0, NCH2 // 2 - 1, cbody, 0)
    pltpu.sync_copy(ones_rows, acc.at[da0], add=True)
    pltpu.sync_copy(ones_rows, acc.at[da1], add=True)
    plsc.subcore_barrier()
    pltpu.sync_copy(acc.at[pl.ds(sid * RPT, RPT)],
                    out_hbm.at[pl.ds(cid * NPAD + sid * RPT, RPT)])


def _deg_kernel(dstp, ones, zeros):
    fn = pl.kernel(
        _deg_body,
        mesh=plsc.VectorSubcoreMesh(core_axis_name="c", subcore_axis_name="s"),
        out_type=jax.ShapeDtypeStruct((NC * NPAD, D), jnp.float32),
        scratch_types=[
            pltpu.VMEM((GCH,), jnp.int32),
            pltpu.VMEM((GCH,), jnp.int32),
            pltpu.VMEM((GCH, D), jnp.float32),
            pltpu.VMEM_SHARED((NPAD, D), jnp.float32),
            pltpu.SemaphoreType.DMA,
            pltpu.SemaphoreType.DMA,
        ],
    )
    return fn(dstp, ones, zeros)


def _agg_body(hs_hbm, srcp_hbm, dstp_hbm, zeros_hbm, out_hbm, sa0, da0, sa1,
              da1, rows0, rows1, acc, sg0, sg1, ss0, ss1):
    cid = lax.axis_index("c")
    sid = lax.axis_index("s")
    wid = sid * NC + cid

    pltpu.sync_copy(zeros_hbm, acc.at[pl.ds(sid * RPT, RPT)])
    plsc.subcore_barrier()

    base0 = wid * NCH2

    pltpu.sync_copy(srcp_hbm.at[base0], sa0)
    pltpu.sync_copy(dstp_hbm.at[base0], da0)
    pltpu.async_copy(hs_hbm.at[sa0], rows0, sg0)
    pltpu.sync_copy(srcp_hbm.at[base0 + 1], sa1)
    pltpu.sync_copy(dstp_hbm.at[base0 + 1], da1)
    pltpu.async_copy(hs_hbm.at[sa1], rows1, sg1)

    def body(g, c):
        k0 = 2 * g
        pltpu.make_async_copy(hs_hbm.at[sa0], rows0, sg0).wait()
        pltpu.async_copy(rows0, acc.at[da0], ss0, add=True)
        pltpu.make_async_copy(hs_hbm.at[sa1], rows1, sg1).wait()
        pltpu.async_copy(rows1, acc.at[da1], ss1, add=True)
        pltpu.make_async_copy(rows0, acc.at[da0], ss0).wait()
        pltpu.sync_copy(srcp_hbm.at[base0 + k0 + 2], sa0)
        pltpu.sync_copy(dstp_hbm.at[base0 + k0 + 2], da0)
        pltpu.async_copy(hs_hbm.at[sa0], rows0, sg0)
        pltpu.make_async_copy(rows1, acc.at[da1], ss1).wait()
        pltpu.sync_copy(srcp_hbm.at[base0 + k0 + 3], sa1)
        pltpu.sync_copy(dstp_hbm.at[base0 + k0 + 3], da1)
        pltpu.async_copy(hs_hbm.at[sa1], rows1, sg1)
        return c

    lax.fori_loop(0, NCH2 // 2 - 1, body, 0)

    pltpu.make_async_copy(hs_hbm.at[sa0], rows0, sg0).wait()
    pltpu.sync_copy(rows0, acc.at[da0], add=True)
    pltpu.make_async_copy(hs_hbm.at[sa1], rows1, sg1).wait()
    pltpu.sync_copy(rows1, acc.at[da1], add=True)

    plsc.subcore_barrier()
    pltpu.sync_copy(acc.at[pl.ds(sid * RPT, RPT)],
                    out_hbm.at[pl.ds(cid * NPAD + sid * RPT, RPT)])


def _agg_kernel(hs, srcp, dstp, zeros):
    fn = pl.kernel(
        _agg_body,
        mesh=plsc.VectorSubcoreMesh(core_axis_name="c", subcore_axis_name="s"),
        out_type=jax.ShapeDtypeStruct((NC * NPAD, D), jnp.float32),
        scratch_types=[
            pltpu.VMEM((GCH,), jnp.int32),
            pltpu.VMEM((GCH,), jnp.int32),
            pltpu.VMEM((GCH,), jnp.int32),
            pltpu.VMEM((GCH,), jnp.int32),
            pltpu.VMEM((GCH, D), jnp.float32),
            pltpu.VMEM((GCH, D), jnp.float32),
            pltpu.VMEM_SHARED((NPAD, D), jnp.float32),
            pltpu.SemaphoreType.DMA,
            pltpu.SemaphoreType.DMA,
            pltpu.SemaphoreType.DMA,
            pltpu.SemaphoreType.DMA,
        ],
    )
    return fn(hs, srcp, dstp, zeros)


def _mm1_body(x_ref, w_ref, dpa_ref, dpb_ref, hs_ref, dv_ref):
    dv = lax.rsqrt(dpa_ref[...] + dpb_ref[...] + 1.0)
    dv_ref[...] = dv
    h = jnp.dot(x_ref[...], w_ref[...], preferred_element_type=jnp.float32)
    hs_ref[...] = h * dv


def _mm1(x, W1, dpa, dpb):
    return pl.pallas_call(
        _mm1_body,
        grid=(NRB,),
        in_specs=[
            pl.BlockSpec((RB, D), lambda i: (i, 0)),
            pl.BlockSpec((D, D), lambda i: (0, 0)),
            pl.BlockSpec((RB, D), lambda i: (i, 0)),
            pl.BlockSpec((RB, D), lambda i: (i, 0)),
        ],
        out_specs=[
            pl.BlockSpec((RB, D), lambda i: (i, 0)),
            pl.BlockSpec((RB, D), lambda i: (i, 0)),
        ],
        out_shape=[
            jax.ShapeDtypeStruct((N, D), jnp.float32),
            jax.ShapeDtypeStruct((N, D), jnp.float32),
        ],
    )(x, W1, dpa, dpb)


def _mid_body(a_ref, b_ref, hs_ref, dv_ref, b1_ref, w2_ref, out_ref):
    t = dv_ref[...] * (a_ref[...] + b_ref[...] + hs_ref[...]) + b1_ref[...]
    h = jnp.maximum(t, 0.0)
    out_ref[...] = jnp.dot(h, w2_ref[...],
                           preferred_element_type=jnp.float32) * dv_ref[...]


def _mid(s1a, s1b, hs1, dinv2, b1, W2):
    return pl.pallas_call(
        _mid_body,
        grid=(NRB,),
        in_specs=[
            pl.BlockSpec((RB, D), lambda i: (i, 0)),
            pl.BlockSpec((RB, D), lambda i: (i, 0)),
            pl.BlockSpec((RB, D), lambda i: (i, 0)),
            pl.BlockSpec((RB, D), lambda i: (i, 0)),
            pl.BlockSpec((1, D), lambda i: (0, 0)),
            pl.BlockSpec((D, D), lambda i: (0, 0)),
        ],
        out_specs=pl.BlockSpec((RB, D), lambda i: (i, 0)),
        out_shape=jax.ShapeDtypeStruct((N, D), jnp.float32),
    )(s1a, s1b, hs1, dinv2, b1, W2)


def _final_body(a_ref, b_ref, hs_ref, dv_ref, b2_ref, bf_ref, wfc_ref, bfc_ref,
                out_ref, sumsT, cnts):
    i = pl.program_id(0)

    @pl.when(i == 0)
    def _():
        sumsT[...] = jnp.zeros((D, G), jnp.float32)
        cnts[...] = jnp.zeros((1, G), jnp.float32)

    t = dv_ref[...] * (a_ref[...] + b_ref[...] + hs_ref[...]) + b2_ref[...]
    h = jnp.maximum(t, 0.0)
    bval = bf_ref[...]
    gid = lax.broadcasted_iota(jnp.int32, (RB, G), 1).astype(jnp.float32)
    oh = jnp.where(bval == gid, 1.0, 0.0)
    sumsT[...] += lax.dot_general(h, oh, (((0,), (0,)), ((), ())),
                                  preferred_element_type=jnp.float32)
    cnts[...] += lax.dot_general(jnp.ones((1, RB), jnp.float32), oh,
                                 (((1,), (0,)), ((), ())),
                                 preferred_element_type=jnp.float32)

    @pl.when(i == NRB - 1)
    def _():
        embT = sumsT[...] / jnp.maximum(cnts[...], 1.0)
        out_ref[...] = lax.dot_general(embT, wfc_ref[...],
                                       (((0,), (0,)), ((), ())),
                                       preferred_element_type=jnp.float32
                                       ) + bfc_ref[...]


def _final(s2a, s2b, hs2, dinv2, b2, batchf, Wfc, bfc):
    return pl.pallas_call(
        _final_body,
        grid=(NRB,),
        in_specs=[
            pl.BlockSpec((RB, D), lambda i: (i, 0)),
            pl.BlockSpec((RB, D), lambda i: (i, 0)),
            pl.BlockSpec((RB, D), lambda i: (i, 0)),
            pl.BlockSpec((RB, D), lambda i: (i, 0)),
            pl.BlockSpec((1, D), lambda i: (0, 0)),
            pl.BlockSpec((RB, G), lambda i: (i, 0)),
            pl.BlockSpec((D, CLS), lambda i: (0, 0)),
            pl.BlockSpec((1, CLS), lambda i: (0, 0)),
        ],
        out_specs=pl.BlockSpec((G, CLS), lambda i: (0, 0)),
        out_shape=jax.ShapeDtypeStruct((G, CLS), jnp.float32),
        scratch_shapes=[
            pltpu.VMEM((D, G), jnp.float32),
            pltpu.VMEM((1, G), jnp.float32),
        ],
    )(s2a, s2b, hs2, dinv2, b2, batchf, Wfc, bfc)


def kernel(x, edge_index, batch, W1, b1, W2, b2, Wfc, bfc):
    ei = edge_index.astype(jnp.int32)
    src = ei[0]
    dst = ei[1]
    batchf = jnp.broadcast_to(batch.astype(jnp.float32)[:, None], (N, G))
    zeros = jnp.zeros((RPT, D), jnp.float32)
    ones = jnp.ones((GCH, D), jnp.float32)
    pad = EPAD - E
    fill = jnp.arange(pad, dtype=jnp.int32)
    srcp = jnp.concatenate([src, fill % N]).reshape(NW * NCH2, GCH)
    dstp = jnp.concatenate([dst, N + fill % (NPAD - N)]).reshape(NW * NCH2, GCH)

    dp = _deg_kernel(dstp, ones, zeros)
    hs1, dinv2 = _mm1(x, W1, dp[:N], dp[NPAD:NPAD + N])
    s1 = _agg_kernel(hs1, srcp, dstp, zeros)
    hs2 = _mid(s1[:N], s1[NPAD:NPAD + N], hs1, dinv2, b1.reshape(1, D), W2)
    s2 = _agg_kernel(hs2, srcp, dstp, zeros)
    return _final(s2[:N], s2[NPAD:NPAD + N], hs2, dinv2, b2.reshape(1, D),
                  batchf, Wfc, bfc.reshape(1, CLS))
